# baseline (speedup 1.0000x reference)
import functools
import math

import jax
import jax.numpy as jnp
from jax import lax
from jax.experimental import pallas as pl
from jax.experimental.pallas import tpu as pltpu
from jax.experimental.pallas import tpu_sc as plsc

D = 64
K = 20
MARGIN = 0.5
NC = 2
NS = 16
NW = NC * NS
CHUNK = 32
L = 16


@functools.lru_cache(maxsize=None)
def _make_transpose_kernel(n_rows: int):
    nblk_full = n_rows // 128
    tail = n_rows - nblk_full * 128
    per_w = nblk_full // NW
    extra = nblk_full % NW
    mesh = plsc.VectorSubcoreMesh(core_axis_name="c", subcore_axis_name="s")

    @functools.partial(
        pl.kernel,
        mesh=mesh,
        compiler_params=pltpu.CompilerParams(
            needs_layout_passes=False, use_tc_tiling_on_sc=True),
        out_type=jax.ShapeDtypeStruct((n_rows // 2, 2 * D), jnp.float32),
        scratch_types=[
            pltpu.VMEM((D, 128), jnp.float32),
            pltpu.VMEM((D, 128), jnp.float32),
            pltpu.VMEM((D, 64), jnp.float32),
            pltpu.SemaphoreType.DMA,
        ],
    )
    def kt(tt_hbm, out_hbm, in_v, pack_v, tin_v, sem):
        wid = lax.axis_index("s") * NC + lax.axis_index("c")
        lanes = lax.iota(jnp.int32, L)
        rowc = [jax.lax.shift_right_logical(lanes + 16 * c, 1)
                for c in range(8)]
        colc = [(lanes & 1) * D for _ in range(8)]

        def do_block(j):
            c0 = pl.multiple_of(j * 128, 128)
            pltpu.async_copy(tt_hbm.at[:, pl.ds(c0, 128)], in_v, sem).wait()

            def d_body(d, _):
                dv = jnp.full((L,), 0, jnp.int32) + d
                for c in range(8):
                    v = in_v[d, pl.ds(16 * c, 16)]
                    plsc.store_scatter(pack_v, [rowc[c], colc[c] + dv], v)
                return 0
            lax.fori_loop(0, D, d_body, 0)
            r0 = pl.multiple_of(j * 64, 8)
            pltpu.sync_copy(pack_v, out_hbm.at[pl.ds(r0, 64), :])

        def blk_body(i, _):
            do_block(wid + i * NW)
            return 0
        lax.fori_loop(0, per_w, blk_body, 0)
        if extra:
            @pl.when(wid < extra)
            def _():
                do_block(wid + per_w * NW)
        if tail:
            @pl.when(wid == NW - 1)
            def _():
                c0 = nblk_full * 128
                pltpu.async_copy(tt_hbm.at[:, pl.ds(c0, tail)],
                                 tin_v, sem).wait()

                def d_body(d, _):
                    dv = jnp.full((L,), 0, jnp.int32) + d
                    for c in range(tail // 16):
                        v = tin_v[d, pl.ds(16 * c, 16)]
                        plsc.store_scatter(
                            pack_v, [rowc[c], colc[c] + dv], v)
                    return 0
                lax.fori_loop(0, D, d_body, 0)
                pltpu.sync_copy(
                    pack_v.at[pl.ds(0, tail // 2), :],
                    out_hbm.at[pl.ds(nblk_full * 64, tail // 2), :])

    return kt


@functools.lru_cache(maxsize=None)
def _make_loss_kernel(batch: int, n_items: int):
    bpw = batch // NW
    nchunks = bpw // CHUNK
    mesh = plsc.VectorSubcoreMesh(core_axis_name="c", subcore_axis_name="s")
    logvals = [math.log(c * n_items / K + 1.0) for c in range(K + 1)]

    @functools.partial(
        pl.kernel,
        mesh=mesh,
        compiler_params=pltpu.CompilerParams(
            needs_layout_passes=False, use_tc_tiling_on_sc=False),
        out_type=jax.ShapeDtypeStruct((NW, 16), jnp.float32),
        scratch_types=[
            pltpu.VMEM((bpw,), jnp.int32),
            pltpu.VMEM((bpw,), jnp.int32),
            pltpu.VMEM((K, bpw), jnp.int32),
            pltpu.VMEM((CHUNK, D), jnp.float32),
            pltpu.VMEM((CHUNK, D), jnp.float32),
            pltpu.VMEM((K, CHUNK, D), jnp.float32),
            pltpu.VMEM((16,), jnp.float32),
            pltpu.SMEM((32,), jnp.float32),
            pltpu.SemaphoreType.DMA,
        ],
    )
    def kb(uid_hbm, pid_hbm, nid_hbm, uemb_hbm, iemb_hbm, out_hbm,
           uid_v, pid_v, nid_v, u_v, p_v, n_v, o_v, logtab, sem):
        wid = lax.axis_index("s") * NC + lax.axis_index("c")
        base = wid * bpw
        pltpu.sync_copy(uid_hbm.at[pl.ds(base, bpw)], uid_v)
        pltpu.sync_copy(pid_hbm.at[pl.ds(base, bpw)], pid_v)
        pltpu.sync_copy(nid_hbm.at[:, pl.ds(base, bpw)], nid_v)
        for c in range(K + 1):
            logtab[c] = jnp.float32(logvals[c])

        def chunk_body(ci, loss):
            off = ci * CHUNK
            cp_u = pltpu.async_copy(uemb_hbm.at[uid_v.at[pl.ds(off, CHUNK)]],
                                    u_v, sem)
            cp_p = pltpu.async_copy(iemb_hbm.at[pid_v.at[pl.ds(off, CHUNK)]],
                                    p_v, sem)
            cps = []
            for k in range(K):
                cps.append(pltpu.async_copy(
                    iemb_hbm.at[nid_v.at[k, pl.ds(off, CHUNK)]],
                    n_v.at[k], sem))
            cp_u.wait()
            cp_p.wait()
            for cp in cps:
                cp.wait()

            def pair_body(b, l):
                uv = [u_v[b, pl.ds(16 * c, 16)] for c in range(4)]
                pv = [p_v[b, pl.ds(16 * c, 16)] for c in range(4)]
                dp0 = uv[0] - pv[0]
                dp1 = uv[1] - pv[1]
                dp2 = uv[2] - pv[2]
                dp3 = uv[3] - pv[3]
                pos_dist = jnp.sum(dp0 * dp0 + dp1 * dp1
                                   + dp2 * dp2 + dp3 * dp3)
                thr = pos_dist + MARGIN
                nds = []
                for k in range(K):
                    d0 = uv[0] - n_v[k, b, pl.ds(0, 16)]
                    d1 = uv[1] - n_v[k, b, pl.ds(16, 16)]
                    d2 = uv[2] - n_v[k, b, pl.ds(32, 16)]
                    d3 = uv[3] - n_v[k, b, pl.ds(48, 16)]
                    nds.append(jnp.sum(d0 * d0 + d1 * d1 + d2 * d2 + d3 * d3))
                closest = functools.reduce(jnp.minimum, nds)
                cnt = jnp.int32(0)
                for nd in nds:
                    cnt = cnt + (thr > nd).astype(jnp.int32)
                lp = jnp.maximum(thr - closest, jnp.float32(0.0))
                return l + lp * logtab[cnt]

            return lax.fori_loop(0, CHUNK, pair_body, loss)

        loss = lax.fori_loop(0, nchunks, chunk_body, jnp.float32(0.0))
        o_v[...] = jnp.broadcast_to(loss, (16,))
        pltpu.sync_copy(o_v, out_hbm.at[wid])

    return kb


def kernel(user_ids, pos_item_ids, neg_item_ids, user_emb, item_emb):
    batch = user_ids.shape[0]
    n_users = user_emb.shape[0]
    n_items = item_emb.shape[0]
    kt = _make_transpose_kernel(n_users)
    u2 = kt(user_emb.T)
    i2 = _make_transpose_kernel(n_items)(item_emb.T)
    kb = _make_loss_kernel(batch, n_items)
    partial = kb(user_ids, pos_item_ids, neg_item_ids.T,
                 u2.reshape(n_users, D), i2.reshape(n_items, D))
    return partial[:, 0].sum()


# speedup vs baseline: 1.2829x; 1.2829x over previous
import functools
import math

import jax
import jax.numpy as jnp
from jax import lax
from jax.experimental import pallas as pl
from jax.experimental.pallas import tpu as pltpu
from jax.experimental.pallas import tpu_sc as plsc

D = 64
K = 20
MARGIN = 0.5
NC = 2
NS = 16
NW = NC * NS
CHUNK = 32
L = 16


@functools.lru_cache(maxsize=None)
def _make_transpose_kernel(n_rows: int):
    nblk_full = n_rows // 128
    tail = n_rows - nblk_full * 128
    per_w = nblk_full // NW
    extra = nblk_full % NW
    mesh = plsc.VectorSubcoreMesh(core_axis_name="c", subcore_axis_name="s")

    @functools.partial(
        pl.kernel,
        mesh=mesh,
        compiler_params=pltpu.CompilerParams(
            needs_layout_passes=False, use_tc_tiling_on_sc=True),
        out_type=jax.ShapeDtypeStruct((n_rows // 2, 2 * D), jnp.float32),
        scratch_types=[
            pltpu.VMEM((2, D, 128), jnp.float32),
            pltpu.VMEM((2, D, 128), jnp.float32),
            pltpu.VMEM((D, 64), jnp.float32),
            pltpu.SemaphoreType.DMA,
            pltpu.SemaphoreType.DMA,
        ],
    )
    def kt(tt_hbm, out_hbm, in_v, pack_v, tin_v, sem_i, sem_o):
        wid = lax.axis_index("s") * NC + lax.axis_index("c")
        lanes = lax.iota(jnp.int32, L)
        rowc = [jax.lax.shift_right_logical(lanes + 16 * c, 1)
                for c in range(8)]
        colc = [(lanes & 1) * D for _ in range(8)]

        def fetch(j, s):
            c0 = pl.multiple_of(j * 128, 128)
            return pltpu.async_copy(tt_hbm.at[:, pl.ds(c0, 128)],
                                    in_v.at[s], sem_i)

        def compute_store(j, s):
            def d_body(d4, _):
                for u in range(4):
                    d = d4 * 4 + u
                    dv = jnp.full((L,), 0, jnp.int32) + d
                    for c in range(8):
                        v = in_v[s, d, pl.ds(16 * c, 16)]
                        plsc.store_scatter(
                            pack_v.at[s], [rowc[c], colc[c] + dv], v)
                return 0
            lax.fori_loop(0, D // 4, d_body, 0)
            r0 = pl.multiple_of(j * 64, 8)
            return pltpu.async_copy(pack_v.at[s], out_hbm.at[pl.ds(r0, 64), :],
                                    sem_o)

        nblk_w = per_w
        fetch(wid, 0)

        def blk_body(i, _):
            s = i % 2
            jn = wid + (i + 1) * NW
            has_next = i + 1 < nblk_w
            @pl.when(has_next)
            def _():
                fetch(jn, (i + 1) % 2)
            pltpu.make_async_copy(tt_hbm.at[:, pl.ds(0, 128)],
                                  in_v.at[s], sem_i).wait()
            @pl.when(i >= 2)
            def _():
                pltpu.make_async_copy(
                    pack_v.at[s],
                    out_hbm.at[pl.ds(pl.multiple_of((wid + (i - 2) * NW) * 64,
                                                    8), 64), :],
                    sem_o).wait()
            compute_store(wid + i * NW, s)
            return 0
        lax.fori_loop(0, nblk_w, blk_body, 0)
        for t in range(2):
            i = nblk_w - 2 + t
            if i >= 0:
                pltpu.make_async_copy(
                    pack_v.at[i % 2],
                    out_hbm.at[pl.ds(
                        pl.multiple_of((wid + i * NW) * 64, 8), 64), :],
                    sem_o).wait()
        if extra:
            @pl.when(wid < extra)
            def _():
                j = wid + per_w * NW
                fetch(j, 0).wait()
                compute_store(j, 0).wait()
        if tail:
            @pl.when(wid == NW - 1)
            def _():
                c0 = nblk_full * 128
                pltpu.async_copy(tt_hbm.at[:, pl.ds(c0, tail)],
                                 tin_v, sem_i).wait()

                def d_body(d, _):
                    dv = jnp.full((L,), 0, jnp.int32) + d
                    for c in range(tail // 16):
                        v = tin_v[d, pl.ds(16 * c, 16)]
                        plsc.store_scatter(
                            pack_v.at[0], [rowc[c], colc[c] + dv], v)
                    return 0
                lax.fori_loop(0, D, d_body, 0)
                pltpu.sync_copy(
                    pack_v.at[0, pl.ds(0, tail // 2), :],
                    out_hbm.at[pl.ds(nblk_full * 64, tail // 2), :])

    return kt


@functools.lru_cache(maxsize=None)
def _make_loss_kernel(batch: int, n_items: int):
    bpw = batch // NW
    nchunks = bpw // CHUNK
    mesh = plsc.VectorSubcoreMesh(core_axis_name="c", subcore_axis_name="s")
    logvals = [math.log(c * n_items / K + 1.0) for c in range(K + 1)]

    @functools.partial(
        pl.kernel,
        mesh=mesh,
        compiler_params=pltpu.CompilerParams(
            needs_layout_passes=False, use_tc_tiling_on_sc=False),
        out_type=jax.ShapeDtypeStruct((NW, 16), jnp.float32),
        scratch_types=[
            pltpu.VMEM((bpw,), jnp.int32),
            pltpu.VMEM((bpw,), jnp.int32),
            pltpu.VMEM((K, bpw), jnp.int32),
            pltpu.VMEM((CHUNK, D), jnp.float32),
            pltpu.VMEM((CHUNK, D), jnp.float32),
            pltpu.VMEM((K, CHUNK, D), jnp.float32),
            pltpu.VMEM((16,), jnp.float32),
            pltpu.SMEM((32,), jnp.float32),
            pltpu.SemaphoreType.DMA,
        ],
    )
    def kb(uid_hbm, pid_hbm, nid_hbm, uemb_hbm, iemb_hbm, out_hbm,
           uid_v, pid_v, nid_v, u_v, p_v, n_v, o_v, logtab, sem):
        wid = lax.axis_index("s") * NC + lax.axis_index("c")
        base = wid * bpw
        pltpu.sync_copy(uid_hbm.at[pl.ds(base, bpw)], uid_v)
        pltpu.sync_copy(pid_hbm.at[pl.ds(base, bpw)], pid_v)
        pltpu.sync_copy(nid_hbm.at[:, pl.ds(base, bpw)], nid_v)
        for c in range(K + 1):
            logtab[c] = jnp.float32(logvals[c])

        def chunk_body(ci, loss):
            off = ci * CHUNK
            cp_u = pltpu.async_copy(uemb_hbm.at[uid_v.at[pl.ds(off, CHUNK)]],
                                    u_v, sem)
            cp_p = pltpu.async_copy(iemb_hbm.at[pid_v.at[pl.ds(off, CHUNK)]],
                                    p_v, sem)
            cps = []
            for k in range(K):
                cps.append(pltpu.async_copy(
                    iemb_hbm.at[nid_v.at[k, pl.ds(off, CHUNK)]],
                    n_v.at[k], sem))
            cp_u.wait()
            cp_p.wait()
            for cp in cps:
                cp.wait()

            def pair_body(b, l):
                uv = [u_v[b, pl.ds(16 * c, 16)] for c in range(4)]
                pv = [p_v[b, pl.ds(16 * c, 16)] for c in range(4)]
                dp0 = uv[0] - pv[0]
                dp1 = uv[1] - pv[1]
                dp2 = uv[2] - pv[2]
                dp3 = uv[3] - pv[3]
                pos_dist = jnp.sum(dp0 * dp0 + dp1 * dp1
                                   + dp2 * dp2 + dp3 * dp3)
                thr = pos_dist + MARGIN
                nds = []
                for k in range(K):
                    d0 = uv[0] - n_v[k, b, pl.ds(0, 16)]
                    d1 = uv[1] - n_v[k, b, pl.ds(16, 16)]
                    d2 = uv[2] - n_v[k, b, pl.ds(32, 16)]
                    d3 = uv[3] - n_v[k, b, pl.ds(48, 16)]
                    nds.append(jnp.sum(d0 * d0 + d1 * d1 + d2 * d2 + d3 * d3))
                closest = functools.reduce(jnp.minimum, nds)
                cnt = jnp.int32(0)
                for nd in nds:
                    cnt = cnt + (thr > nd).astype(jnp.int32)
                lp = jnp.maximum(thr - closest, jnp.float32(0.0))
                return l + lp * logtab[cnt]

            return lax.fori_loop(0, CHUNK, pair_body, loss)

        loss = lax.fori_loop(0, nchunks, chunk_body, jnp.float32(0.0))
        o_v[...] = jnp.broadcast_to(loss, (16,))
        pltpu.sync_copy(o_v, out_hbm.at[wid])

    return kb


def kernel(user_ids, pos_item_ids, neg_item_ids, user_emb, item_emb):
    batch = user_ids.shape[0]
    n_users = user_emb.shape[0]
    n_items = item_emb.shape[0]
    kt = _make_transpose_kernel(n_users)
    u2 = kt(user_emb.T)
    i2 = _make_transpose_kernel(n_items)(item_emb.T)
    kb = _make_loss_kernel(batch, n_items)
    partial = kb(user_ids, pos_item_ids, neg_item_ids.T,
                 u2.reshape(n_users, D), i2.reshape(n_items, D))
    return partial[:, 0].sum()


# speedup vs baseline: 1.9981x; 1.5574x over previous
import functools
import math

import jax
import jax.numpy as jnp
from jax import lax
from jax.experimental import pallas as pl
from jax.experimental.pallas import tpu as pltpu
from jax.experimental.pallas import tpu_sc as plsc

D = 64
K = 20
MARGIN = 0.5
NC = 2
NS = 16
NW = NC * NS
CHUNK = 32
L = 16


@functools.lru_cache(maxsize=None)
def _make_transpose_kernel(n_rows: int):
    nblk_full = n_rows // 128
    tail = n_rows - nblk_full * 128
    per_w = nblk_full // NW
    extra = nblk_full % NW
    mesh = plsc.VectorSubcoreMesh(core_axis_name="c", subcore_axis_name="s")

    @functools.partial(
        pl.kernel,
        mesh=mesh,
        compiler_params=pltpu.CompilerParams(
            needs_layout_passes=False, use_tc_tiling_on_sc=True),
        out_type=jax.ShapeDtypeStruct((n_rows // 2, 2 * D), jnp.float32),
        scratch_types=[
            pltpu.VMEM((2, D, 128), jnp.float32),
            pltpu.VMEM((2, D, 128), jnp.float32),
            pltpu.VMEM((D, 64), jnp.float32),
            pltpu.SemaphoreType.DMA,
            pltpu.SemaphoreType.DMA,
        ],
    )
    def kt(tt_hbm, out_hbm, in_v, pack_v, tin_v, sem_i, sem_o):
        wid = lax.axis_index("s") * NC + lax.axis_index("c")
        lanes = lax.iota(jnp.int32, L)
        rowc = [jax.lax.shift_right_logical(lanes + 16 * c, 1)
                for c in range(8)]
        colc = [(lanes & 1) * D for _ in range(8)]

        def fetch(j, s):
            c0 = pl.multiple_of(j * 128, 128)
            return pltpu.async_copy(tt_hbm.at[:, pl.ds(c0, 128)],
                                    in_v.at[s], sem_i)

        def compute_store(j, s):
            sb = jnp.full((L,), 0, jnp.int32) + s

            def dg_body(dg, _):
                dvec = lanes + dg * L
                for jg in range(8):
                    for t in range(L):
                        jvec = jg * L + ((lanes + t) & (L - 1))
                        v = plsc.load_gather(in_v, [sb, dvec, jvec])
                        rsc = jax.lax.shift_right_logical(jvec, 1)
                        csc = (jvec & 1) * D + dvec
                        plsc.store_scatter(pack_v, [sb, rsc, csc], v)
                return 0
            lax.fori_loop(0, D // L, dg_body, 0)
            r0 = pl.multiple_of(j * 64, 8)
            return pltpu.async_copy(pack_v.at[s], out_hbm.at[pl.ds(r0, 64), :],
                                    sem_o)

        nblk_w = per_w
        fetch(wid, 0)

        def blk_body(i, _):
            s = i % 2
            jn = wid + (i + 1) * NW
            has_next = i + 1 < nblk_w
            @pl.when(has_next)
            def _():
                fetch(jn, (i + 1) % 2)
            pltpu.make_async_copy(tt_hbm.at[:, pl.ds(0, 128)],
                                  in_v.at[s], sem_i).wait()
            @pl.when(i >= 2)
            def _():
                pltpu.make_async_copy(
                    pack_v.at[s],
                    out_hbm.at[pl.ds(pl.multiple_of((wid + (i - 2) * NW) * 64,
                                                    8), 64), :],
                    sem_o).wait()
            compute_store(wid + i * NW, s)
            return 0
        lax.fori_loop(0, nblk_w, blk_body, 0)
        for t in range(2):
            i = nblk_w - 2 + t
            if i >= 0:
                pltpu.make_async_copy(
                    pack_v.at[i % 2],
                    out_hbm.at[pl.ds(
                        pl.multiple_of((wid + i * NW) * 64, 8), 64), :],
                    sem_o).wait()
        if extra:
            @pl.when(wid < extra)
            def _():
                j = wid + per_w * NW
                fetch(j, 0).wait()
                compute_store(j, 0).wait()
        if tail:
            @pl.when(wid == NW - 1)
            def _():
                c0 = nblk_full * 128
                pltpu.async_copy(tt_hbm.at[:, pl.ds(c0, tail)],
                                 tin_v, sem_i).wait()

                def d_body(d, _):
                    dv = jnp.full((L,), 0, jnp.int32) + d
                    for c in range(tail // 16):
                        v = tin_v[d, pl.ds(16 * c, 16)]
                        plsc.store_scatter(
                            pack_v.at[0], [rowc[c], colc[c] + dv], v)
                    return 0
                lax.fori_loop(0, D, d_body, 0)
                pltpu.sync_copy(
                    pack_v.at[0, pl.ds(0, tail // 2), :],
                    out_hbm.at[pl.ds(nblk_full * 64, tail // 2), :])

    return kt


@functools.lru_cache(maxsize=None)
def _make_loss_kernel(batch: int, n_items: int):
    bpw = batch // NW
    nchunks = bpw // CHUNK
    mesh = plsc.VectorSubcoreMesh(core_axis_name="c", subcore_axis_name="s")
    logvals = [math.log(c * n_items / K + 1.0) for c in range(K + 1)]

    @functools.partial(
        pl.kernel,
        mesh=mesh,
        compiler_params=pltpu.CompilerParams(
            needs_layout_passes=False, use_tc_tiling_on_sc=False),
        out_type=jax.ShapeDtypeStruct((NW, 16), jnp.float32),
        scratch_types=[
            pltpu.VMEM((bpw,), jnp.int32),
            pltpu.VMEM((bpw,), jnp.int32),
            pltpu.VMEM((K, bpw), jnp.int32),
            pltpu.VMEM((CHUNK, D), jnp.float32),
            pltpu.VMEM((CHUNK, D), jnp.float32),
            pltpu.VMEM((K, CHUNK, D), jnp.float32),
            pltpu.VMEM((16,), jnp.float32),
            pltpu.SMEM((32,), jnp.float32),
            pltpu.SemaphoreType.DMA,
        ],
    )
    def kb(uid_hbm, pid_hbm, nid_hbm, uemb_hbm, iemb_hbm, out_hbm,
           uid_v, pid_v, nid_v, u_v, p_v, n_v, o_v, logtab, sem):
        wid = lax.axis_index("s") * NC + lax.axis_index("c")
        base = wid * bpw
        pltpu.sync_copy(uid_hbm.at[pl.ds(base, bpw)], uid_v)
        pltpu.sync_copy(pid_hbm.at[pl.ds(base, bpw)], pid_v)
        pltpu.sync_copy(nid_hbm.at[:, pl.ds(base, bpw)], nid_v)
        for c in range(K + 1):
            logtab[c] = jnp.float32(logvals[c])

        def chunk_body(ci, loss):
            off = ci * CHUNK
            cp_u = pltpu.async_copy(uemb_hbm.at[uid_v.at[pl.ds(off, CHUNK)]],
                                    u_v, sem)
            cp_p = pltpu.async_copy(iemb_hbm.at[pid_v.at[pl.ds(off, CHUNK)]],
                                    p_v, sem)
            cps = []
            for k in range(K):
                cps.append(pltpu.async_copy(
                    iemb_hbm.at[nid_v.at[k, pl.ds(off, CHUNK)]],
                    n_v.at[k], sem))
            cp_u.wait()
            cp_p.wait()
            for cp in cps:
                cp.wait()

            def pair_body(b, l):
                uv = [u_v[b, pl.ds(16 * c, 16)] for c in range(4)]
                pv = [p_v[b, pl.ds(16 * c, 16)] for c in range(4)]
                dp0 = uv[0] - pv[0]
                dp1 = uv[1] - pv[1]
                dp2 = uv[2] - pv[2]
                dp3 = uv[3] - pv[3]
                pos_dist = jnp.sum(dp0 * dp0 + dp1 * dp1
                                   + dp2 * dp2 + dp3 * dp3)
                thr = pos_dist + MARGIN
                nds = []
                for k in range(K):
                    d0 = uv[0] - n_v[k, b, pl.ds(0, 16)]
                    d1 = uv[1] - n_v[k, b, pl.ds(16, 16)]
                    d2 = uv[2] - n_v[k, b, pl.ds(32, 16)]
                    d3 = uv[3] - n_v[k, b, pl.ds(48, 16)]
                    nds.append(jnp.sum(d0 * d0 + d1 * d1 + d2 * d2 + d3 * d3))
                closest = functools.reduce(jnp.minimum, nds)
                cnt = jnp.int32(0)
                for nd in nds:
                    cnt = cnt + (thr > nd).astype(jnp.int32)
                lp = jnp.maximum(thr - closest, jnp.float32(0.0))
                return l + lp * logtab[cnt]

            return lax.fori_loop(0, CHUNK, pair_body, loss)

        loss = lax.fori_loop(0, nchunks, chunk_body, jnp.float32(0.0))
        o_v[...] = jnp.broadcast_to(loss, (16,))
        pltpu.sync_copy(o_v, out_hbm.at[wid])

    return kb


def kernel(user_ids, pos_item_ids, neg_item_ids, user_emb, item_emb):
    batch = user_ids.shape[0]
    n_users = user_emb.shape[0]
    n_items = item_emb.shape[0]
    kt = _make_transpose_kernel(n_users)
    u2 = kt(user_emb.T)
    i2 = _make_transpose_kernel(n_items)(item_emb.T)
    kb = _make_loss_kernel(batch, n_items)
    partial = kb(user_ids, pos_item_ids, neg_item_ids.T,
                 u2.reshape(n_users, D), i2.reshape(n_items, D))
    return partial[:, 0].sum()


# speedup vs baseline: 2.9784x; 1.4906x over previous
import functools
import math

import jax
import jax.numpy as jnp
from jax import lax
from jax.experimental import pallas as pl
from jax.experimental.pallas import tpu as pltpu
from jax.experimental.pallas import tpu_sc as plsc

D = 64
K = 20
MARGIN = 0.5
NC = 2
NS = 16
NW = NC * NS
CHUNK = 32
L = 16


@functools.lru_cache(maxsize=None)
def _make_transpose_kernel(n_rows: int):
    nblk_full = n_rows // 128
    tail = n_rows - nblk_full * 128
    per_w = nblk_full // NW
    extra = nblk_full % NW
    mesh = plsc.VectorSubcoreMesh(core_axis_name="c", subcore_axis_name="s")

    @functools.partial(
        pl.kernel,
        mesh=mesh,
        compiler_params=pltpu.CompilerParams(
            needs_layout_passes=False, use_tc_tiling_on_sc=True),
        out_type=jax.ShapeDtypeStruct((n_rows // 2, 2 * D), jnp.float32),
        scratch_types=[
            pltpu.VMEM((2, D, 128), jnp.float32),
            pltpu.VMEM((2, D, 128), jnp.float32),
            pltpu.VMEM((D, 64), jnp.float32),
            pltpu.SemaphoreType.DMA,
            pltpu.SemaphoreType.DMA,
        ],
    )
    def kt(tt_hbm, out_hbm, in_v, pack_v, tin_v, sem_i, sem_o):
        wid = lax.axis_index("s") * NC + lax.axis_index("c")
        lanes = lax.iota(jnp.int32, L)
        rowc = [jax.lax.shift_right_logical(lanes + 16 * c, 1)
                for c in range(8)]
        colc = [(lanes & 1) * D for _ in range(8)]

        def fetch(j, s):
            c0 = pl.multiple_of(j * 128, 128)
            return pltpu.async_copy(tt_hbm.at[:, pl.ds(c0, 128)],
                                    in_v.at[s], sem_i)

        def compute_store(j, s):
            sb = jnp.full((L,), 0, jnp.int32) + s
            dvecs = [lanes + dg * L for dg in range(D // L)]

            def jg_body(jg, _):
                for t in range(L):
                    jvec = jg * L + ((lanes + t) & (L - 1))
                    rsc = jax.lax.shift_right_logical(jvec, 1)
                    cscb = (jvec & 1) * D
                    for dg in range(D // L):
                        v = plsc.load_gather(in_v, [sb, dvecs[dg], jvec])
                        plsc.store_scatter(
                            pack_v, [sb, rsc, cscb + dvecs[dg]], v)
                return 0
            lax.fori_loop(0, 8, jg_body, 0)
            r0 = pl.multiple_of(j * 64, 8)
            return pltpu.async_copy(pack_v.at[s], out_hbm.at[pl.ds(r0, 64), :],
                                    sem_o)

        nblk_w = per_w
        fetch(wid, 0)

        def blk_body(i, _):
            s = i % 2
            jn = wid + (i + 1) * NW
            has_next = i + 1 < nblk_w
            @pl.when(has_next)
            def _():
                fetch(jn, (i + 1) % 2)
            pltpu.make_async_copy(tt_hbm.at[:, pl.ds(0, 128)],
                                  in_v.at[s], sem_i).wait()
            @pl.when(i >= 2)
            def _():
                pltpu.make_async_copy(
                    pack_v.at[s],
                    out_hbm.at[pl.ds(pl.multiple_of((wid + (i - 2) * NW) * 64,
                                                    8), 64), :],
                    sem_o).wait()
            compute_store(wid + i * NW, s)
            return 0
        lax.fori_loop(0, nblk_w, blk_body, 0)
        for t in range(2):
            i = nblk_w - 2 + t
            if i >= 0:
                pltpu.make_async_copy(
                    pack_v.at[i % 2],
                    out_hbm.at[pl.ds(
                        pl.multiple_of((wid + i * NW) * 64, 8), 64), :],
                    sem_o).wait()
        if extra:
            @pl.when(wid < extra)
            def _():
                j = wid + per_w * NW
                fetch(j, 0).wait()
                compute_store(j, 0).wait()
        if tail:
            @pl.when(wid == NW - 1)
            def _():
                c0 = nblk_full * 128
                pltpu.async_copy(tt_hbm.at[:, pl.ds(c0, tail)],
                                 tin_v, sem_i).wait()

                def d_body(d, _):
                    dv = jnp.full((L,), 0, jnp.int32) + d
                    for c in range(tail // 16):
                        v = tin_v[d, pl.ds(16 * c, 16)]
                        plsc.store_scatter(
                            pack_v.at[0], [rowc[c], colc[c] + dv], v)
                    return 0
                lax.fori_loop(0, D, d_body, 0)
                pltpu.sync_copy(
                    pack_v.at[0, pl.ds(0, tail // 2), :],
                    out_hbm.at[pl.ds(nblk_full * 64, tail // 2), :])

    return kt


@functools.lru_cache(maxsize=None)
def _make_loss_kernel(batch: int, n_items: int):
    bpw = batch // NW
    nchunks = bpw // CHUNK
    mesh = plsc.VectorSubcoreMesh(core_axis_name="c", subcore_axis_name="s")
    logvals = [math.log(c * n_items / K + 1.0) for c in range(K + 1)]

    @functools.partial(
        pl.kernel,
        mesh=mesh,
        compiler_params=pltpu.CompilerParams(
            needs_layout_passes=False, use_tc_tiling_on_sc=False),
        out_type=jax.ShapeDtypeStruct((NW, 16), jnp.float32),
        scratch_types=[
            pltpu.VMEM((bpw,), jnp.int32),
            pltpu.VMEM((bpw,), jnp.int32),
            pltpu.VMEM((K, bpw), jnp.int32),
            pltpu.VMEM((CHUNK, D), jnp.float32),
            pltpu.VMEM((CHUNK, D), jnp.float32),
            pltpu.VMEM((K, CHUNK, D), jnp.float32),
            pltpu.VMEM((16,), jnp.float32),
            pltpu.SMEM((32,), jnp.float32),
            pltpu.SemaphoreType.DMA,
        ],
    )
    def kb(uid_hbm, pid_hbm, nid_hbm, uemb_hbm, iemb_hbm, out_hbm,
           uid_v, pid_v, nid_v, u_v, p_v, n_v, o_v, logtab, sem):
        wid = lax.axis_index("s") * NC + lax.axis_index("c")
        base = wid * bpw
        pltpu.sync_copy(uid_hbm.at[pl.ds(base, bpw)], uid_v)
        pltpu.sync_copy(pid_hbm.at[pl.ds(base, bpw)], pid_v)
        pltpu.sync_copy(nid_hbm.at[:, pl.ds(base, bpw)], nid_v)
        for c in range(K + 1):
            logtab[c] = jnp.float32(logvals[c])

        def chunk_body(ci, loss):
            off = ci * CHUNK
            cp_u = pltpu.async_copy(uemb_hbm.at[uid_v.at[pl.ds(off, CHUNK)]],
                                    u_v, sem)
            cp_p = pltpu.async_copy(iemb_hbm.at[pid_v.at[pl.ds(off, CHUNK)]],
                                    p_v, sem)
            cps = []
            for k in range(K):
                cps.append(pltpu.async_copy(
                    iemb_hbm.at[nid_v.at[k, pl.ds(off, CHUNK)]],
                    n_v.at[k], sem))
            cp_u.wait()
            cp_p.wait()
            for cp in cps:
                cp.wait()

            def pair_body(b, l):
                uv = [u_v[b, pl.ds(16 * c, 16)] for c in range(4)]
                pv = [p_v[b, pl.ds(16 * c, 16)] for c in range(4)]
                dp0 = uv[0] - pv[0]
                dp1 = uv[1] - pv[1]
                dp2 = uv[2] - pv[2]
                dp3 = uv[3] - pv[3]
                pos_dist = jnp.sum(dp0 * dp0 + dp1 * dp1
                                   + dp2 * dp2 + dp3 * dp3)
                thr = pos_dist + MARGIN
                nds = []
                for k in range(K):
                    d0 = uv[0] - n_v[k, b, pl.ds(0, 16)]
                    d1 = uv[1] - n_v[k, b, pl.ds(16, 16)]
                    d2 = uv[2] - n_v[k, b, pl.ds(32, 16)]
                    d3 = uv[3] - n_v[k, b, pl.ds(48, 16)]
                    nds.append(jnp.sum(d0 * d0 + d1 * d1 + d2 * d2 + d3 * d3))
                closest = functools.reduce(jnp.minimum, nds)
                cnt = jnp.int32(0)
                for nd in nds:
                    cnt = cnt + (thr > nd).astype(jnp.int32)
                lp = jnp.maximum(thr - closest, jnp.float32(0.0))
                return l + lp * logtab[cnt]

            return lax.fori_loop(0, CHUNK, pair_body, loss)

        loss = lax.fori_loop(0, nchunks, chunk_body, jnp.float32(0.0))
        o_v[...] = jnp.broadcast_to(loss, (16,))
        pltpu.sync_copy(o_v, out_hbm.at[wid])

    return kb


def kernel(user_ids, pos_item_ids, neg_item_ids, user_emb, item_emb):
    batch = user_ids.shape[0]
    n_users = user_emb.shape[0]
    n_items = item_emb.shape[0]
    u2 = _make_transpose_kernel(n_users)(user_emb.T)
    i2 = _make_transpose_kernel(n_items)(item_emb.T)
    kb = _make_loss_kernel(batch, n_items)
    partial = kb(user_ids, pos_item_ids, neg_item_ids.T,
                 u2.reshape(n_users, D), i2.reshape(n_items, D))
    return partial[:, 0].sum()


# speedup vs baseline: 3.2453x; 1.0896x over previous
import functools
import math

import jax
import jax.numpy as jnp
from jax import lax
from jax.experimental import pallas as pl
from jax.experimental.pallas import tpu as pltpu
from jax.experimental.pallas import tpu_sc as plsc

D = 64
K = 20
MARGIN = 0.5
NC = 2
NS = 16
NW = NC * NS
CHUNK = 32
TCHUNK = 64
L = 16


@functools.lru_cache(maxsize=None)
def _make_transpose_kernel(n_rows: int):
    nblk_full = n_rows // 128
    tail = n_rows - nblk_full * 128
    per_w = nblk_full // NW
    extra = nblk_full % NW
    mesh = plsc.VectorSubcoreMesh(core_axis_name="c", subcore_axis_name="s")

    @functools.partial(
        pl.kernel,
        mesh=mesh,
        compiler_params=pltpu.CompilerParams(
            needs_layout_passes=False, use_tc_tiling_on_sc=True),
        out_type=jax.ShapeDtypeStruct((n_rows // 2, 2 * D), jnp.float32),
        scratch_types=[
            pltpu.VMEM((2, D, 128), jnp.float32),
            pltpu.VMEM((2, D, 128), jnp.float32),
            pltpu.VMEM((D, 64), jnp.float32),
            pltpu.SemaphoreType.DMA,
            pltpu.SemaphoreType.DMA,
        ],
    )
    def kt(tt_hbm, out_hbm, in_v, pack_v, tin_v, sem_i, sem_o):
        wid = lax.axis_index("s") * NC + lax.axis_index("c")
        lanes = lax.iota(jnp.int32, L)
        rowc = [jax.lax.shift_right_logical(lanes + 16 * c, 1)
                for c in range(8)]
        colc = [(lanes & 1) * D for _ in range(8)]

        def fetch(j, s):
            c0 = pl.multiple_of(j * 128, 128)
            return pltpu.async_copy(tt_hbm.at[:, pl.ds(c0, 128)],
                                    in_v.at[s], sem_i)

        def compute_store(j, s):
            sb = jnp.full((L,), 0, jnp.int32) + s
            dvecs = [lanes + dg * L for dg in range(D // L)]

            def jg_body(jg, _):
                for t in range(L):
                    jvec = jg * L + ((lanes + t) & (L - 1))
                    rsc = jax.lax.shift_right_logical(jvec, 1)
                    cscb = (jvec & 1) * D
                    for dg in range(D // L):
                        v = plsc.load_gather(in_v, [sb, dvecs[dg], jvec])
                        plsc.store_scatter(
                            pack_v, [sb, rsc, cscb + dvecs[dg]], v)
                return 0
            lax.fori_loop(0, 8, jg_body, 0)
            r0 = pl.multiple_of(j * 64, 8)
            return pltpu.async_copy(pack_v.at[s], out_hbm.at[pl.ds(r0, 64), :],
                                    sem_o)

        nblk_w = per_w
        fetch(wid, 0)

        def blk_body(i, _):
            s = i % 2
            jn = wid + (i + 1) * NW
            has_next = i + 1 < nblk_w
            @pl.when(has_next)
            def _():
                fetch(jn, (i + 1) % 2)
            pltpu.make_async_copy(tt_hbm.at[:, pl.ds(0, 128)],
                                  in_v.at[s], sem_i).wait()
            @pl.when(i >= 2)
            def _():
                pltpu.make_async_copy(
                    pack_v.at[s],
                    out_hbm.at[pl.ds(pl.multiple_of((wid + (i - 2) * NW) * 64,
                                                    8), 64), :],
                    sem_o).wait()
            compute_store(wid + i * NW, s)
            return 0
        lax.fori_loop(0, nblk_w, blk_body, 0)
        for t in range(2):
            i = nblk_w - 2 + t
            if i >= 0:
                pltpu.make_async_copy(
                    pack_v.at[i % 2],
                    out_hbm.at[pl.ds(
                        pl.multiple_of((wid + i * NW) * 64, 8), 64), :],
                    sem_o).wait()
        if extra:
            @pl.when(wid < extra)
            def _():
                j = wid + per_w * NW
                fetch(j, 0).wait()
                compute_store(j, 0).wait()
        if tail:
            @pl.when(wid == NW - 1)
            def _():
                c0 = nblk_full * 128
                pltpu.async_copy(tt_hbm.at[:, pl.ds(c0, tail)],
                                 tin_v, sem_i).wait()

                def d_body(d, _):
                    dv = jnp.full((L,), 0, jnp.int32) + d
                    for c in range(tail // 16):
                        v = tin_v[d, pl.ds(16 * c, 16)]
                        plsc.store_scatter(
                            pack_v.at[0], [rowc[c], colc[c] + dv], v)
                    return 0
                lax.fori_loop(0, D, d_body, 0)
                pltpu.sync_copy(
                    pack_v.at[0, pl.ds(0, tail // 2), :],
                    out_hbm.at[pl.ds(nblk_full * 64, tail // 2), :])

    return kt


@functools.lru_cache(maxsize=None)
def _make_user_rows_kernel(batch: int, n_users: int):
    bpw = batch // NW
    nchunks = bpw // TCHUNK
    mesh = plsc.VectorSubcoreMesh(core_axis_name="c", subcore_axis_name="s")

    @functools.partial(
        pl.kernel,
        mesh=mesh,
        compiler_params=pltpu.CompilerParams(
            needs_layout_passes=False, use_tc_tiling_on_sc=True),
        out_type=jax.ShapeDtypeStruct((batch // 2, 2 * D), jnp.float32),
        scratch_types=[
            pltpu.VMEM((bpw,), jnp.int32),
            pltpu.VMEM((bpw,), jnp.int32),
            pltpu.VMEM((TCHUNK, 8, D), jnp.float32),
            pltpu.VMEM((TCHUNK // 2, 2 * D), jnp.float32),
            pltpu.SemaphoreType.DMA,
        ],
    )
    def ka(uid_hbm, utab_hbm, out_hbm, uid_v, trow_v, tiles_v, pack_v, sem):
        wid = lax.axis_index("s") * NC + lax.axis_index("c")
        base = wid * bpw
        pltpu.sync_copy(uid_hbm.at[pl.ds(base, bpw)], uid_v)

        def rows_body(i, _):
            s = pl.ds(i * L, L)
            trow_v[s] = jax.lax.shift_right_logical(uid_v[s], 3)
            return 0
        lax.fori_loop(0, bpw // L, rows_body, 0)

        lanes = lax.iota(jnp.int32, L)

        def chunk_body(ci, _):
            off = ci * TCHUNK
            cps = []
            for g4 in range(TCHUNK // L):
                tv = trow_v[pl.ds(off + g4 * L, L)]
                for j in range(L):
                    t = jnp.max(jnp.where(lanes == j, tv, jnp.int32(0)))
                    t8 = pl.multiple_of(t * 8, 8)
                    cps.append(pltpu.async_copy(
                        utab_hbm.at[pl.ds(t8, 8), :],
                        tiles_v.at[g4 * L + j], sem))
            for cp in cps:
                cp.wait()
            for g in range(TCHUNK // L):
                pvec = lanes + g * L
                subrow = uid_v[pl.ds(off + g * L, L)] & 7
                dst_r = jax.lax.shift_right_logical(pvec, 1)
                dst_c0 = (pvec & 1) * D

                def d_body(d, _):
                    dv = jnp.full((L,), 0, jnp.int32) + d
                    val = plsc.load_gather(tiles_v, [pvec, subrow, dv])
                    plsc.store_scatter(pack_v, [dst_r, dst_c0 + dv], val)
                    return 0
                lax.fori_loop(0, D, d_body, 0)
            r0 = pl.multiple_of((base + off) // 2, 8)
            pltpu.sync_copy(
                pack_v, out_hbm.at[pl.ds(r0, TCHUNK // 2), :])
            return 0

        lax.fori_loop(0, nchunks, chunk_body, 0)

    return ka


@functools.lru_cache(maxsize=None)
def _make_loss_kernel(batch: int, n_items: int):
    bpw = batch // NW
    nchunks = bpw // CHUNK
    mesh = plsc.VectorSubcoreMesh(core_axis_name="c", subcore_axis_name="s")
    logvals = [math.log(c * n_items / K + 1.0) for c in range(K + 1)]

    @functools.partial(
        pl.kernel,
        mesh=mesh,
        compiler_params=pltpu.CompilerParams(
            needs_layout_passes=False, use_tc_tiling_on_sc=False),
        out_type=jax.ShapeDtypeStruct((NW, 16), jnp.float32),
        scratch_types=[
            pltpu.VMEM((bpw // 2, 2 * D), jnp.float32),
            pltpu.VMEM((bpw,), jnp.int32),
            pltpu.VMEM((K, bpw), jnp.int32),
            pltpu.VMEM((CHUNK, D), jnp.float32),
            pltpu.VMEM((K, CHUNK, D), jnp.float32),
            pltpu.VMEM((16,), jnp.float32),
            pltpu.SMEM((32,), jnp.float32),
            pltpu.SemaphoreType.DMA,
        ],
    )
    def kb(upack_hbm, pid_hbm, nid_hbm, iemb_hbm, out_hbm,
           u_v, pid_v, nid_v, p_v, n_v, o_v, logtab, sem):
        wid = lax.axis_index("s") * NC + lax.axis_index("c")
        base = wid * bpw
        pltpu.sync_copy(
            upack_hbm.at[pl.ds(pl.multiple_of(base // 2, 8), bpw // 2), :],
            u_v)
        pltpu.sync_copy(pid_hbm.at[pl.ds(base, bpw)], pid_v)
        pltpu.sync_copy(nid_hbm.at[:, pl.ds(base, bpw)], nid_v)
        for c in range(K + 1):
            logtab[c] = jnp.float32(logvals[c])

        def chunk_body(ci, loss):
            off = ci * CHUNK
            cp_p = pltpu.async_copy(iemb_hbm.at[pid_v.at[pl.ds(off, CHUNK)]],
                                    p_v, sem)
            cps = []
            for k in range(K):
                cps.append(pltpu.async_copy(
                    iemb_hbm.at[nid_v.at[k, pl.ds(off, CHUNK)]],
                    n_v.at[k], sem))
            cp_p.wait()
            for cp in cps:
                cp.wait()

            def pair_body(b, l):
                gb = off + b
                ur = jax.lax.shift_right_logical(gb, 1)
                uc = (gb & 1) * D
                uv = [u_v[ur, pl.ds(uc + 16 * c, 16)] for c in range(4)]
                pv = [p_v[b, pl.ds(16 * c, 16)] for c in range(4)]
                dp0 = uv[0] - pv[0]
                dp1 = uv[1] - pv[1]
                dp2 = uv[2] - pv[2]
                dp3 = uv[3] - pv[3]
                pos_dist = jnp.sum(dp0 * dp0 + dp1 * dp1
                                   + dp2 * dp2 + dp3 * dp3)
                thr = pos_dist + MARGIN
                nds = []
                for k in range(K):
                    d0 = uv[0] - n_v[k, b, pl.ds(0, 16)]
                    d1 = uv[1] - n_v[k, b, pl.ds(16, 16)]
                    d2 = uv[2] - n_v[k, b, pl.ds(32, 16)]
                    d3 = uv[3] - n_v[k, b, pl.ds(48, 16)]
                    nds.append(jnp.sum(d0 * d0 + d1 * d1 + d2 * d2 + d3 * d3))
                closest = functools.reduce(jnp.minimum, nds)
                cnt = jnp.int32(0)
                for nd in nds:
                    cnt = cnt + (thr > nd).astype(jnp.int32)
                lp = jnp.maximum(thr - closest, jnp.float32(0.0))
                return l + lp * logtab[cnt]

            return lax.fori_loop(0, CHUNK, pair_body, loss)

        loss = lax.fori_loop(0, nchunks, chunk_body, jnp.float32(0.0))
        o_v[...] = jnp.broadcast_to(loss, (16,))
        pltpu.sync_copy(o_v, out_hbm.at[wid])

    return kb


def kernel(user_ids, pos_item_ids, neg_item_ids, user_emb, item_emb):
    batch = user_ids.shape[0]
    n_users = user_emb.shape[0]
    n_items = item_emb.shape[0]
    upack = _make_user_rows_kernel(batch, n_users)(user_ids, user_emb)
    i2 = _make_transpose_kernel(n_items)(item_emb.T)
    kb = _make_loss_kernel(batch, n_items)
    partial = kb(upack, pos_item_ids, neg_item_ids.T, i2.reshape(n_items, D))
    return partial[:, 0].sum()


# speedup vs baseline: 3.2465x; 1.0004x over previous
import functools
import math

import jax
import jax.numpy as jnp
from jax import lax
from jax.experimental import pallas as pl
from jax.experimental.pallas import tpu as pltpu
from jax.experimental.pallas import tpu_sc as plsc

D = 64
K = 20
MARGIN = 0.5
NC = 2
NS = 16
NW = NC * NS
CHUNK = 32
TCHUNK = 64
L = 16


@functools.lru_cache(maxsize=None)
def _make_transpose_kernel(n_rows: int):
    nblk_full = n_rows // 128
    tail = n_rows - nblk_full * 128
    per_w = nblk_full // NW
    extra = nblk_full % NW
    mesh = plsc.VectorSubcoreMesh(core_axis_name="c", subcore_axis_name="s")

    @functools.partial(
        pl.kernel,
        mesh=mesh,
        compiler_params=pltpu.CompilerParams(
            needs_layout_passes=False, use_tc_tiling_on_sc=True),
        out_type=jax.ShapeDtypeStruct((n_rows // 2, 2 * D), jnp.float32),
        scratch_types=[
            pltpu.VMEM((2, D, 128), jnp.float32),
            pltpu.VMEM((2, D, 128), jnp.float32),
            pltpu.VMEM((D, 64), jnp.float32),
            pltpu.SemaphoreType.DMA,
            pltpu.SemaphoreType.DMA,
        ],
    )
    def kt(tt_hbm, out_hbm, in_v, pack_v, tin_v, sem_i, sem_o):
        wid = lax.axis_index("s") * NC + lax.axis_index("c")
        lanes = lax.iota(jnp.int32, L)
        rowc = [jax.lax.shift_right_logical(lanes + 16 * c, 1)
                for c in range(8)]
        colc = [(lanes & 1) * D for _ in range(8)]

        def fetch(j, s):
            c0 = pl.multiple_of(j * 128, 128)
            return pltpu.async_copy(tt_hbm.at[:, pl.ds(c0, 128)],
                                    in_v.at[s], sem_i)

        def compute_store(j, s):
            sb = jnp.full((L,), 0, jnp.int32) + s
            dvecs = [lanes + dg * L for dg in range(D // L)]

            def jg_body(jg, _):
                for t in range(L):
                    jvec = jg * L + ((lanes + t) & (L - 1))
                    rsc = jax.lax.shift_right_logical(jvec, 1)
                    cscb = (jvec & 1) * D
                    for dg in range(D // L):
                        v = plsc.load_gather(in_v, [sb, dvecs[dg], jvec])
                        plsc.store_scatter(
                            pack_v, [sb, rsc, cscb + dvecs[dg]], v)
                return 0
            lax.fori_loop(0, 8, jg_body, 0)
            r0 = pl.multiple_of(j * 64, 8)
            return pltpu.async_copy(pack_v.at[s], out_hbm.at[pl.ds(r0, 64), :],
                                    sem_o)

        nblk_w = per_w
        fetch(wid, 0)

        def blk_body(i, _):
            s = i % 2
            jn = wid + (i + 1) * NW
            has_next = i + 1 < nblk_w
            @pl.when(has_next)
            def _():
                fetch(jn, (i + 1) % 2)
            pltpu.make_async_copy(tt_hbm.at[:, pl.ds(0, 128)],
                                  in_v.at[s], sem_i).wait()
            @pl.when(i >= 2)
            def _():
                pltpu.make_async_copy(
                    pack_v.at[s],
                    out_hbm.at[pl.ds(pl.multiple_of((wid + (i - 2) * NW) * 64,
                                                    8), 64), :],
                    sem_o).wait()
            compute_store(wid + i * NW, s)
            return 0
        lax.fori_loop(0, nblk_w, blk_body, 0)
        for t in range(2):
            i = nblk_w - 2 + t
            if i >= 0:
                pltpu.make_async_copy(
                    pack_v.at[i % 2],
                    out_hbm.at[pl.ds(
                        pl.multiple_of((wid + i * NW) * 64, 8), 64), :],
                    sem_o).wait()
        if extra:
            @pl.when(wid < extra)
            def _():
                j = wid + per_w * NW
                fetch(j, 0).wait()
                compute_store(j, 0).wait()
        if tail:
            @pl.when(wid == NW - 1)
            def _():
                c0 = nblk_full * 128
                pltpu.async_copy(tt_hbm.at[:, pl.ds(c0, tail)],
                                 tin_v, sem_i).wait()

                def d_body(d, _):
                    dv = jnp.full((L,), 0, jnp.int32) + d
                    for c in range(tail // 16):
                        v = tin_v[d, pl.ds(16 * c, 16)]
                        plsc.store_scatter(
                            pack_v.at[0], [rowc[c], colc[c] + dv], v)
                    return 0
                lax.fori_loop(0, D, d_body, 0)
                pltpu.sync_copy(
                    pack_v.at[0, pl.ds(0, tail // 2), :],
                    out_hbm.at[pl.ds(nblk_full * 64, tail // 2), :])

    return kt


@functools.lru_cache(maxsize=None)
def _make_user_rows_kernel(batch: int, n_users: int):
    bpw = batch // NW
    nchunks = bpw // TCHUNK
    mesh = plsc.VectorSubcoreMesh(core_axis_name="c", subcore_axis_name="s")

    @functools.partial(
        pl.kernel,
        mesh=mesh,
        compiler_params=pltpu.CompilerParams(
            needs_layout_passes=False, use_tc_tiling_on_sc=True),
        out_type=jax.ShapeDtypeStruct((batch // 2, 2 * D), jnp.float32),
        scratch_types=[
            pltpu.VMEM((bpw,), jnp.int32),
            pltpu.VMEM((bpw,), jnp.int32),
            pltpu.VMEM((TCHUNK, 8, D), jnp.float32),
            pltpu.VMEM((TCHUNK // 2, 2 * D), jnp.float32),
            pltpu.SemaphoreType.DMA,
        ],
    )
    def ka(uid_hbm, utab_hbm, out_hbm, uid_v, trow_v, tiles_v, pack_v, sem):
        wid = lax.axis_index("s") * NC + lax.axis_index("c")
        base = wid * bpw
        pltpu.sync_copy(uid_hbm.at[pl.ds(base, bpw)], uid_v)

        def rows_body(i, _):
            s = pl.ds(i * L, L)
            trow_v[s] = jax.lax.shift_right_logical(uid_v[s], 3)
            return 0
        lax.fori_loop(0, bpw // L, rows_body, 0)

        lanes = lax.iota(jnp.int32, L)

        def chunk_body(ci, _):
            off = ci * TCHUNK
            cps = []
            for g4 in range(TCHUNK // L):
                tv = trow_v[pl.ds(off + g4 * L, L)]
                for j in range(L):
                    t = jnp.max(jnp.where(lanes == j, tv, jnp.int32(0)))
                    t8 = pl.multiple_of(t * 8, 8)
                    cps.append(pltpu.async_copy(
                        utab_hbm.at[pl.ds(t8, 8), :],
                        tiles_v.at[g4 * L + j], sem))
            for cp in cps:
                cp.wait()
            for g in range(TCHUNK // L):
                pvec = lanes + g * L
                subrow = uid_v[pl.ds(off + g * L, L)] & 7
                dst_r = jax.lax.shift_right_logical(pvec, 1)
                dst_c0 = (pvec & 1) * D

                def d_body(d, _):
                    dv = jnp.full((L,), 0, jnp.int32) + d
                    val = plsc.load_gather(tiles_v, [pvec, subrow, dv])
                    plsc.store_scatter(pack_v, [dst_r, dst_c0 + dv], val)
                    return 0
                lax.fori_loop(0, D, d_body, 0)
            r0 = pl.multiple_of((base + off) // 2, 8)
            pltpu.sync_copy(
                pack_v, out_hbm.at[pl.ds(r0, TCHUNK // 2), :])
            return 0

        lax.fori_loop(0, nchunks, chunk_body, 0)

    return ka


@functools.lru_cache(maxsize=None)
def _make_loss_kernel(batch: int, n_items: int):
    bpw = batch // NW
    nchunks = bpw // CHUNK
    mesh = plsc.VectorSubcoreMesh(core_axis_name="c", subcore_axis_name="s")
    logvals = [math.log(c * n_items / K + 1.0) for c in range(K + 1)]

    @functools.partial(
        pl.kernel,
        mesh=mesh,
        compiler_params=pltpu.CompilerParams(
            needs_layout_passes=False, use_tc_tiling_on_sc=False),
        out_type=jax.ShapeDtypeStruct((NW, 16), jnp.float32),
        scratch_types=[
            pltpu.VMEM((bpw // 2, 2 * D), jnp.float32),
            pltpu.VMEM((bpw,), jnp.int32),
            pltpu.VMEM((K, bpw), jnp.int32),
            pltpu.VMEM((CHUNK, D), jnp.float32),
            pltpu.VMEM((K, CHUNK, D), jnp.float32),
            pltpu.VMEM((16,), jnp.float32),
            pltpu.SMEM((32,), jnp.float32),
            pltpu.SemaphoreType.DMA,
        ],
    )
    def kb(upack_hbm, pid_hbm, nid_hbm, iemb_hbm, out_hbm,
           u_v, pid_v, nid_v, p_v, n_v, o_v, logtab, sem):
        wid = lax.axis_index("s") * NC + lax.axis_index("c")
        base = wid * bpw
        pltpu.sync_copy(
            upack_hbm.at[pl.ds(pl.multiple_of(base // 2, 8), bpw // 2), :],
            u_v)
        pltpu.sync_copy(pid_hbm.at[pl.ds(base, bpw)], pid_v)
        pltpu.sync_copy(nid_hbm.at[:, pl.ds(base, bpw)], nid_v)
        for c in range(K + 1):
            logtab[c] = jnp.float32(logvals[c])

        def chunk_body(ci, loss):
            off = ci * CHUNK
            cp_p = pltpu.async_copy(iemb_hbm.at[pid_v.at[pl.ds(off, CHUNK)]],
                                    p_v, sem)
            cps = []
            for k in range(K):
                cps.append(pltpu.async_copy(
                    iemb_hbm.at[nid_v.at[k, pl.ds(off, CHUNK)]],
                    n_v.at[k], sem))
            cp_p.wait()
            for cp in cps:
                cp.wait()

            def pair_body(b, l):
                gb = off + b
                ur = jax.lax.shift_right_logical(gb, 1)
                uc = (gb & 1) * D
                uv = [u_v[ur, pl.ds(uc + 16 * c, 16)] for c in range(4)]
                pv = [p_v[b, pl.ds(16 * c, 16)] for c in range(4)]
                dp0 = uv[0] - pv[0]
                dp1 = uv[1] - pv[1]
                dp2 = uv[2] - pv[2]
                dp3 = uv[3] - pv[3]
                pos_dist = jnp.sum(dp0 * dp0 + dp1 * dp1
                                   + dp2 * dp2 + dp3 * dp3)
                thr = pos_dist + MARGIN
                nds = []
                for k in range(K):
                    d0 = uv[0] - n_v[k, b, pl.ds(0, 16)]
                    d1 = uv[1] - n_v[k, b, pl.ds(16, 16)]
                    d2 = uv[2] - n_v[k, b, pl.ds(32, 16)]
                    d3 = uv[3] - n_v[k, b, pl.ds(48, 16)]
                    nds.append(jnp.sum(d0 * d0 + d1 * d1 + d2 * d2 + d3 * d3))
                closest = functools.reduce(jnp.minimum, nds)
                cnt = jnp.int32(0)
                for nd in nds:
                    cnt = cnt + (thr > nd).astype(jnp.int32)
                lp = jnp.maximum(thr - closest, jnp.float32(0.0))
                return l + lp * logtab[cnt]

            return lax.fori_loop(0, CHUNK, pair_body, loss)

        loss = lax.fori_loop(0, nchunks, chunk_body, jnp.float32(0.0))
        o_v[...] = jnp.broadcast_to(loss, (16,))
        pltpu.sync_copy(o_v, out_hbm.at[wid])

    return kb


def kernel(user_ids, pos_item_ids, neg_item_ids, user_emb, item_emb):
    batch = user_ids.shape[0]
    n_users = user_emb.shape[0]
    n_items = item_emb.shape[0]
    i2 = _make_transpose_kernel(n_items)(item_emb.T)
    upack = _make_user_rows_kernel(batch, n_users)(user_ids, user_emb)
    kb = _make_loss_kernel(batch, n_items)
    partial = kb(upack, pos_item_ids, neg_item_ids.T, i2.reshape(n_items, D))
    return partial[:, 0].sum()


# speedup vs baseline: 3.3873x; 1.0434x over previous
import functools
import math

import jax
import jax.numpy as jnp
from jax import lax
from jax.experimental import pallas as pl
from jax.experimental.pallas import tpu as pltpu
from jax.experimental.pallas import tpu_sc as plsc

D = 64
K = 20
MARGIN = 0.5
NC = 2
NS = 16
NW = NC * NS
CHUNK = 32
TCHUNK = 64
L = 16


@functools.lru_cache(maxsize=None)
def _make_user_rows_kernel(batch: int, n_users: int):
    bpw = batch // NW
    nchunks = bpw // TCHUNK
    mesh = plsc.VectorSubcoreMesh(core_axis_name="c", subcore_axis_name="s")

    @functools.partial(
        pl.kernel,
        mesh=mesh,
        compiler_params=pltpu.CompilerParams(
            needs_layout_passes=False, use_tc_tiling_on_sc=True),
        out_type=jax.ShapeDtypeStruct((batch // 2, 2 * D), jnp.float32),
        scratch_types=[
            pltpu.VMEM((bpw,), jnp.int32),
            pltpu.VMEM((bpw,), jnp.int32),
            pltpu.VMEM((TCHUNK, 8, D), jnp.float32),
            pltpu.VMEM((TCHUNK // 2, 2 * D), jnp.float32),
            pltpu.SemaphoreType.DMA,
        ],
    )
    def ka(uid_hbm, utab_hbm, out_hbm, uid_v, trow_v, tiles_v, pack_v, sem):
        wid = lax.axis_index("s") * NC + lax.axis_index("c")
        base = wid * bpw
        pltpu.sync_copy(uid_hbm.at[pl.ds(base, bpw)], uid_v)

        def rows_body(i, _):
            s = pl.ds(i * L, L)
            trow_v[s] = jax.lax.shift_right_logical(uid_v[s], 3)
            return 0
        lax.fori_loop(0, bpw // L, rows_body, 0)

        lanes = lax.iota(jnp.int32, L)

        def chunk_body(ci, _):
            off = ci * TCHUNK
            cps = []
            for g4 in range(TCHUNK // L):
                tv = trow_v[pl.ds(off + g4 * L, L)]
                for j in range(L):
                    t = jnp.max(jnp.where(lanes == j, tv, jnp.int32(0)))
                    t8 = pl.multiple_of(t * 8, 8)
                    cps.append(pltpu.async_copy(
                        utab_hbm.at[pl.ds(t8, 8), :],
                        tiles_v.at[g4 * L + j], sem))
            for cp in cps:
                cp.wait()
            for g in range(TCHUNK // L):
                pvec = lanes + g * L
                subrow = uid_v[pl.ds(off + g * L, L)] & 7
                dst_r = jax.lax.shift_right_logical(pvec, 1)
                dst_c0 = (pvec & 1) * D

                def d_body(d, _):
                    dv = jnp.full((L,), 0, jnp.int32) + d
                    val = plsc.load_gather(tiles_v, [pvec, subrow, dv])
                    plsc.store_scatter(pack_v, [dst_r, dst_c0 + dv], val)
                    return 0
                lax.fori_loop(0, D, d_body, 0)
            r0 = pl.multiple_of((base + off) // 2, 8)
            pltpu.sync_copy(
                pack_v, out_hbm.at[pl.ds(r0, TCHUNK // 2), :])
            return 0

        lax.fori_loop(0, nchunks, chunk_body, 0)

    return ka


@functools.lru_cache(maxsize=None)
def _make_loss_kernel(batch: int, n_items: int):
    bpw = batch // NW
    nchunks = bpw // CHUNK
    mesh = plsc.VectorSubcoreMesh(core_axis_name="c", subcore_axis_name="s")
    logvals = [math.log(c * n_items / K + 1.0) for c in range(K + 1)]

    @functools.partial(
        pl.kernel,
        mesh=mesh,
        compiler_params=pltpu.CompilerParams(
            needs_layout_passes=False, use_tc_tiling_on_sc=False),
        out_type=jax.ShapeDtypeStruct((NW, 16), jnp.float32),
        scratch_types=[
            pltpu.VMEM((bpw // 2, 2 * D), jnp.float32),
            pltpu.VMEM((bpw,), jnp.int32),
            pltpu.VMEM((K, bpw), jnp.int32),
            pltpu.VMEM((CHUNK, D), jnp.float32),
            pltpu.VMEM((K, CHUNK, D), jnp.float32),
            pltpu.VMEM((16,), jnp.float32),
            pltpu.SMEM((32,), jnp.float32),
            pltpu.SemaphoreType.DMA,
        ],
    )
    def kb(upack_hbm, pid_hbm, nid_hbm, iemb_hbm, out_hbm,
           u_v, pid_v, nid_v, p_v, n_v, o_v, logtab, sem):
        wid = lax.axis_index("s") * NC + lax.axis_index("c")
        base = wid * bpw
        pltpu.sync_copy(
            upack_hbm.at[pl.ds(pl.multiple_of(base // 2, 8), bpw // 2), :],
            u_v)
        pltpu.sync_copy(pid_hbm.at[pl.ds(base, bpw)], pid_v)
        pltpu.sync_copy(nid_hbm.at[:, pl.ds(base, bpw)], nid_v)
        for c in range(K + 1):
            logtab[c] = jnp.float32(logvals[c])

        def chunk_body(ci, loss):
            off = ci * CHUNK
            cp_p = pltpu.async_copy(iemb_hbm.at[pid_v.at[pl.ds(off, CHUNK)]],
                                    p_v, sem)
            cps = []
            for k in range(K):
                cps.append(pltpu.async_copy(
                    iemb_hbm.at[nid_v.at[k, pl.ds(off, CHUNK)]],
                    n_v.at[k], sem))
            cp_p.wait()
            for cp in cps:
                cp.wait()

            def pair_body(b, l):
                gb = off + b
                ur = jax.lax.shift_right_logical(gb, 1)
                uc = (gb & 1) * D
                uv = [u_v[ur, pl.ds(uc + 16 * c, 16)] for c in range(4)]
                pv = [p_v[b, pl.ds(16 * c, 16)] for c in range(4)]
                dp0 = uv[0] - pv[0]
                dp1 = uv[1] - pv[1]
                dp2 = uv[2] - pv[2]
                dp3 = uv[3] - pv[3]
                pos_dist = jnp.sum(dp0 * dp0 + dp1 * dp1
                                   + dp2 * dp2 + dp3 * dp3)
                thr = pos_dist + MARGIN
                nds = []
                for k in range(K):
                    d0 = uv[0] - n_v[k, b, pl.ds(0, 16)]
                    d1 = uv[1] - n_v[k, b, pl.ds(16, 16)]
                    d2 = uv[2] - n_v[k, b, pl.ds(32, 16)]
                    d3 = uv[3] - n_v[k, b, pl.ds(48, 16)]
                    nds.append(jnp.sum(d0 * d0 + d1 * d1 + d2 * d2 + d3 * d3))
                closest = functools.reduce(jnp.minimum, nds)
                cnt = jnp.int32(0)
                for nd in nds:
                    cnt = cnt + (thr > nd).astype(jnp.int32)
                lp = jnp.maximum(thr - closest, jnp.float32(0.0))
                return l + lp * logtab[cnt]

            return lax.fori_loop(0, CHUNK, pair_body, loss)

        loss = lax.fori_loop(0, nchunks, chunk_body, jnp.float32(0.0))
        o_v[...] = jnp.broadcast_to(loss, (16,))
        pltpu.sync_copy(o_v, out_hbm.at[wid])

    return kb


def kernel(user_ids, pos_item_ids, neg_item_ids, user_emb, item_emb):
    batch = user_ids.shape[0]
    n_users = user_emb.shape[0]
    n_items = item_emb.shape[0]
    ka = _make_user_rows_kernel(batch, n_users)
    upack = ka(user_ids, user_emb)
    kb = _make_loss_kernel(batch, n_items)
    partial = kb(upack, pos_item_ids, neg_item_ids.T, item_emb)
    return partial[:, 0].sum()
